# baseline (speedup 1.0000x reference)
import functools

import jax
import jax.numpy as jnp
from jax import lax
from jax.experimental import pallas as pl
from jax.experimental.pallas import tpu as pltpu
from jax.experimental.pallas import tpu_sc as plsc

_VOCAB = 1000000
_EMBED_DIM = 16
_BATCH = 16384

_NUM_CORES = 2
_NUM_SUBCORES = 16
_NUM_WORKERS = _NUM_CORES * _NUM_SUBCORES
_B_PER_W = _BATCH // _NUM_WORKERS


@functools.partial(
    pl.kernel,
    mesh=plsc.VectorSubcoreMesh(core_axis_name="c", subcore_axis_name="s"),
    out_type=jax.ShapeDtypeStruct((_EMBED_DIM, _BATCH), jnp.float32),
    scratch_types=[
        pltpu.VMEM((_B_PER_W,), jnp.int32),
        pltpu.VMEM((_EMBED_DIM, _B_PER_W), jnp.float32),
        pltpu.SemaphoreType.DMA,
    ],
    compiler_params=pltpu.CompilerParams(use_tc_tiling_on_sc=False),
)
def _gather_kernel(tableT_hbm, idx_hbm, out_hbm, idx_v, rows_v, sem):
    wid = lax.axis_index("s") * _NUM_CORES + lax.axis_index("c")
    base = wid * _B_PER_W
    pltpu.sync_copy(idx_hbm.at[pl.ds(base, _B_PER_W)], idx_v)
    for f in range(_EMBED_DIM):
        pltpu.async_copy(tableT_hbm.at[f].at[idx_v], rows_v.at[f], sem)
    for f in range(_EMBED_DIM):
        pltpu.make_async_copy(
            tableT_hbm.at[f].at[pl.ds(0, _B_PER_W)], rows_v.at[f], sem
        ).wait()
    pltpu.sync_copy(rows_v, out_hbm.at[:, pl.ds(base, _B_PER_W)])


def kernel(inputs, table):
    outT = _gather_kernel(table.T, inputs.astype(jnp.int32))
    return outT.T


# speedup vs baseline: 1.7343x; 1.7343x over previous
import functools

import jax
import jax.numpy as jnp
from jax import lax
from jax.experimental import pallas as pl
from jax.experimental.pallas import tpu as pltpu
from jax.experimental.pallas import tpu_sc as plsc

_VOCAB = 1000000
_EMBED_DIM = 16
_BATCH = 16384

_NUM_CORES = 2
_NUM_SUBCORES = 16
_NUM_WORKERS = _NUM_CORES * _NUM_SUBCORES
_B_PER_W = _BATCH // _NUM_WORKERS

_TR_BLOCK = 2048


def _transpose_block(inT_ref, out_ref):
    out_ref[...] = inT_ref[...].T


_transpose = pl.pallas_call(
    _transpose_block,
    grid=(pl.cdiv(_VOCAB, _TR_BLOCK),),
    in_specs=[pl.BlockSpec((_EMBED_DIM, _TR_BLOCK), lambda i: (0, i))],
    out_specs=pl.BlockSpec((_TR_BLOCK, _EMBED_DIM), lambda i: (i, 0)),
    out_shape=jax.ShapeDtypeStruct((_VOCAB, _EMBED_DIM), jnp.float32),
)


@functools.partial(
    pl.kernel,
    mesh=plsc.VectorSubcoreMesh(core_axis_name="c", subcore_axis_name="s"),
    out_type=jax.ShapeDtypeStruct((_BATCH, _EMBED_DIM), jnp.float32),
    scratch_types=[
        pltpu.VMEM((_B_PER_W,), jnp.int32),
        pltpu.VMEM((_B_PER_W, _EMBED_DIM), jnp.float32),
        pltpu.SemaphoreType.DMA,
    ],
    compiler_params=pltpu.CompilerParams(use_tc_tiling_on_sc=False),
)
def _gather_kernel(table_hbm, idx_hbm, out_hbm, idx_v, rows_v, sem):
    wid = lax.axis_index("s") * _NUM_CORES + lax.axis_index("c")
    base = wid * _B_PER_W
    pltpu.sync_copy(idx_hbm.at[pl.ds(base, _B_PER_W)], idx_v)
    pltpu.async_copy(table_hbm.at[idx_v], rows_v, sem).wait()
    pltpu.sync_copy(rows_v, out_hbm.at[pl.ds(base, _B_PER_W)])


def kernel(inputs, table):
    table_rm = _transpose(table.T)
    return _gather_kernel(table_rm, inputs.astype(jnp.int32))


# speedup vs baseline: 2.6791x; 1.5447x over previous
import functools

import jax
import jax.numpy as jnp
from jax import lax
from jax.experimental import pallas as pl
from jax.experimental.pallas import tpu as pltpu
from jax.experimental.pallas import tpu_sc as plsc

_VOCAB = 1000000
_EMBED_DIM = 16
_BATCH = 16384

_NUM_CORES = 2
_NUM_SUBCORES = 16
_NUM_WORKERS = _NUM_CORES * _NUM_SUBCORES
_B_PER_W = _BATCH // _NUM_WORKERS

_CHUNK = 2048
_FULL_CHUNKS = _VOCAB // _CHUNK
_REM = _VOCAB - _FULL_CHUNKS * _CHUNK
_TAIL1 = 512
_TAIL1_OFF = _FULL_CHUNKS * _CHUNK
_TAIL2 = _REM - _TAIL1
_TAIL2_OFF = _TAIL1_OFF + _TAIL1
_ROUNDS = (_FULL_CHUNKS + _NUM_WORKERS - 1) // _NUM_WORKERS


@functools.partial(
    pl.kernel,
    mesh=plsc.VectorSubcoreMesh(core_axis_name="c", subcore_axis_name="s"),
    out_type=jax.ShapeDtypeStruct((_VOCAB * _EMBED_DIM,), jnp.float32),
    scratch_types=[
        pltpu.VMEM((_EMBED_DIM, _CHUNK), jnp.float32),
        pltpu.VMEM((_CHUNK * _EMBED_DIM,), jnp.float32),
    ],
    compiler_params=pltpu.CompilerParams(needs_layout_passes=False),
)
def _transpose_kernel(tableT_hbm, tailf_hbm, outf_hbm, in_v, flat_v):
    wid = lax.axis_index("s") * _NUM_CORES + lax.axis_index("c")
    rows = lax.iota(jnp.int32, _EMBED_DIM)

    def do_chunk(col0, width):
        pltpu.sync_copy(
            tableT_hbm.at[:, pl.ds(col0, width)], in_v.at[:, pl.ds(0, width)]
        )

        def body(j, carry):
            vals = plsc.load_gather(in_v, [rows, jnp.full((_EMBED_DIM,), j, jnp.int32)])
            flat_v[pl.ds(j * _EMBED_DIM, _EMBED_DIM)] = vals
            return carry

        lax.fori_loop(0, width, body, 0)
        pltpu.sync_copy(
            flat_v.at[pl.ds(0, width * _EMBED_DIM)],
            outf_hbm.at[pl.ds(col0 * _EMBED_DIM, width * _EMBED_DIM)],
        )

    def round_body(k, carry):
        chunk = wid + k * _NUM_WORKERS

        @pl.when(chunk < _FULL_CHUNKS)
        def _():
            do_chunk(chunk * _CHUNK, _CHUNK)

        return carry

    lax.fori_loop(0, _ROUNDS, round_body, 0)

    @pl.when(wid == _NUM_WORKERS - 2)
    def _():
        do_chunk(_TAIL1_OFF, _TAIL1)

    @pl.when(wid == _NUM_WORKERS - 1)
    def _():
        pltpu.sync_copy(tailf_hbm, flat_v.at[pl.ds(0, _TAIL2 * _EMBED_DIM)])
        pltpu.sync_copy(
            flat_v.at[pl.ds(0, _TAIL2 * _EMBED_DIM)],
            outf_hbm.at[pl.ds(_TAIL2_OFF * _EMBED_DIM, _TAIL2 * _EMBED_DIM)],
        )


@functools.partial(
    pl.kernel,
    mesh=plsc.VectorSubcoreMesh(core_axis_name="c", subcore_axis_name="s"),
    out_type=jax.ShapeDtypeStruct((_BATCH, _EMBED_DIM), jnp.float32),
    scratch_types=[
        pltpu.VMEM((_B_PER_W,), jnp.int32),
        pltpu.VMEM((_B_PER_W, _EMBED_DIM), jnp.float32),
        pltpu.SemaphoreType.DMA,
    ],
    compiler_params=pltpu.CompilerParams(use_tc_tiling_on_sc=False),
)
def _gather_kernel(table_hbm, idx_hbm, out_hbm, idx_v, rows_v, sem):
    wid = lax.axis_index("s") * _NUM_CORES + lax.axis_index("c")
    base = wid * _B_PER_W
    pltpu.sync_copy(idx_hbm.at[pl.ds(base, _B_PER_W)], idx_v)
    pltpu.async_copy(table_hbm.at[idx_v], rows_v, sem).wait()
    pltpu.sync_copy(rows_v, out_hbm.at[pl.ds(base, _B_PER_W)])


def kernel(inputs, table):
    tail_flat = lax.slice(
        table, (_TAIL2_OFF, 0), (_VOCAB, _EMBED_DIM)
    ).reshape(_TAIL2 * _EMBED_DIM)
    table_rm = _transpose_kernel(table.T, tail_flat).reshape(_VOCAB, _EMBED_DIM)
    return _gather_kernel(table_rm, inputs.astype(jnp.int32))
